# Initial kernel scaffold; baseline (speedup 1.0000x reference)
import jax
import jax.numpy as jnp
from jax.experimental import pallas as pl


def kernel(beta, gamma, A, Z_i, Z_j, G, sampling_i_weights, sampling_j_weights, sparse_i_idx, sparse_j_idx):
    raise NotImplementedError("write your pallas kernel here")


# speedup vs baseline: 1.1068x; 1.1068x over previous
import jax
import jax.numpy as jnp
from jax.experimental import pallas as pl

_N_I = 100000
_N_J = 100000
_K = 8
_E = 200000
_S_I = 1000
_S_J = 1000
_EPS = 1e-6
_EBLK = 8192
_EPAD = 204800


def _dense_kernel(zi_ref, zj_ref, gt_ref, a_ref, zsi_ref, zsj_ref,
                  bsi_ref, gsj_ref, azc_ref, mat_ref):
    zi = jax.nn.softmax(zi_ref[...], axis=0)
    zj = jax.nn.softmax(zj_ref[...], axis=0)
    z = jnp.concatenate([zi, zj], axis=1)
    ztg = z * jax.nn.sigmoid(gt_ref[...])
    c = ztg / jnp.sum(ztg, axis=1, keepdims=True)
    zc = jax.lax.dot_general(z, c, (((1,), (1,)), ((), ())),
                             preferred_element_type=jnp.float32)
    azc = jnp.dot(a_ref[...], zc, preferred_element_type=jnp.float32).T
    azc_ref[...] = azc
    mi = jnp.dot(azc, jax.nn.softmax(zsi_ref[...], axis=0),
                 preferred_element_type=jnp.float32) + _EPS
    mj = jnp.dot(azc, jax.nn.softmax(zsj_ref[...], axis=0),
                 preferred_element_type=jnp.float32)
    rowsq = jnp.sum(mi * mi, axis=0, keepdims=True)
    colsq = jnp.sum(mj * mj, axis=0, keepdims=True)
    u = jnp.concatenate([mi, jnp.ones_like(rowsq), rowsq], axis=0)
    v = jnp.concatenate([-2.0 * mj, colsq, jnp.ones_like(colsq)], axis=0)
    d2 = jax.lax.dot_general(u, v, (((0,), (0,)), ((), ())),
                             preferred_element_type=jnp.float32)
    dist = jnp.sqrt(jnp.maximum(d2, 0.0))
    mat_ref[...] = jnp.sum(jnp.exp(bsi_ref[...] + gsj_ref[...] - dist),
                           keepdims=True)


def _edge_kernel(azc_ref, zie_ref, zje_ref, be_ref, ge_ref, keep_ref, out_ref):
    i = pl.program_id(0)

    @pl.when(i == 0)
    def _init():
        out_ref[...] = jnp.zeros_like(out_ref)

    azc = azc_ref[...]
    li = jnp.dot(azc, jax.nn.softmax(zie_ref[...], axis=0),
                 preferred_element_type=jnp.float32)
    lj = jnp.dot(azc, jax.nn.softmax(zje_ref[...], axis=0),
                 preferred_element_type=jnp.float32)
    diff = li - lj + _EPS
    sq = jnp.sum(diff * diff, axis=0, keepdims=True)
    link = be_ref[...] + ge_ref[...] - sq
    out_ref[...] += jnp.sum(link * keep_ref[...], keepdims=True)


def kernel(beta, gamma, A, Z_i, Z_j, G, sampling_i_weights, sampling_j_weights,
           sparse_i_idx, sparse_j_idx):
    key = jax.random.key(42)
    k1, k2 = jax.random.split(key)
    u1 = jax.random.uniform(k1, sampling_i_weights.shape, minval=1e-12, maxval=1.0)
    g1 = -jnp.log(-jnp.log(u1))
    _, si = jax.lax.top_k(jnp.log(sampling_i_weights + 1e-20) + g1, _S_I)
    u2 = jax.random.uniform(k2, sampling_j_weights.shape, minval=1e-12, maxval=1.0)
    g2 = -jnp.log(-jnp.log(u2))
    _, sj = jax.lax.top_k(jnp.log(sampling_j_weights + 1e-20) + g2, _S_J)

    mask_i = jnp.zeros((_N_I,), dtype=bool).at[si].set(True)
    mask_j = jnp.zeros((_N_J,), dtype=bool).at[sj].set(True)
    keep = (mask_i[sparse_i_idx] & mask_j[sparse_j_idx]).astype(jnp.float32)

    gt = G.T
    zsi = jnp.take(Z_i, si, axis=1)
    zsj = jnp.take(Z_j, sj, axis=1)
    bsi = beta[si].reshape(_S_I, 1)
    gsj = gamma[sj].reshape(1, _S_J)

    zie = jnp.take(Z_i, sparse_i_idx, axis=1)
    zje = jnp.take(Z_j, sparse_j_idx, axis=1)
    be = beta[sparse_i_idx].reshape(1, _E)
    ge = gamma[sparse_j_idx].reshape(1, _E)

    pad = _EPAD - _E
    zie = jnp.pad(zie, ((0, 0), (0, pad)))
    zje = jnp.pad(zje, ((0, 0), (0, pad)))
    be = jnp.pad(be, ((0, 0), (0, pad)))
    ge = jnp.pad(ge, ((0, 0), (0, pad)))
    keep = jnp.pad(keep.reshape(1, _E), ((0, 0), (0, pad)))

    azc, mat = pl.pallas_call(
        _dense_kernel,
        out_shape=[
            jax.ShapeDtypeStruct((_K, _K), jnp.float32),
            jax.ShapeDtypeStruct((1, 1), jnp.float32),
        ],
    )(Z_i, Z_j, gt, A, zsi, zsj, bsi, gsj)

    nblk = _EPAD // _EBLK
    links = pl.pallas_call(
        _edge_kernel,
        grid=(nblk,),
        in_specs=[
            pl.BlockSpec((_K, _K), lambda i: (0, 0)),
            pl.BlockSpec((_K, _EBLK), lambda i: (0, i)),
            pl.BlockSpec((_K, _EBLK), lambda i: (0, i)),
            pl.BlockSpec((1, _EBLK), lambda i: (0, i)),
            pl.BlockSpec((1, _EBLK), lambda i: (0, i)),
            pl.BlockSpec((1, _EBLK), lambda i: (0, i)),
        ],
        out_specs=pl.BlockSpec((1, 1), lambda i: (0, 0)),
        out_shape=jax.ShapeDtypeStruct((1, 1), jnp.float32),
    )(azc, zie, zje, be, ge, keep)

    return links[0, 0] - mat[0, 0]
